# baseline (speedup 1.0000x reference)
import functools
import math

import jax
import jax.numpy as jnp
from jax import lax
from jax.experimental import pallas as pl
from jax.experimental.pallas import tpu as pltpu
from jax.experimental.pallas import tpu_sc as plsc

_SMOOTHING = 0.1
_CONFIDENCE = 1.0 - _SMOOTHING

_NUM_CORES = 2
_NUM_SUBCORES = 16
_NW = _NUM_CORES * _NUM_SUBCORES
_L = 16

_C = 1408
_NACC = 8


def _sc_partials(x, tgt, aligned, s_coef, g_coef):
    n, v = x.shape
    rpt = n // _NW
    nchunk = aligned // _C
    npair = nchunk // 2
    ngrp = _C // _L
    mesh = plsc.VectorSubcoreMesh(core_axis_name="c", subcore_axis_name="s")

    @functools.partial(
        pl.kernel,
        mesh=mesh,
        out_type=jax.ShapeDtypeStruct((_NW, _L), jnp.float32),
        scratch_types=[
            pltpu.VMEM((rpt, _C), jnp.float32),
            pltpu.VMEM((rpt, _C), jnp.float32),
            pltpu.VMEM((rpt,), jnp.int32),
            pltpu.VMEM((_L,), jnp.float32),
            pltpu.SemaphoreType.DMA,
            pltpu.SemaphoreType.DMA,
        ],
        compiler_params=pltpu.CompilerParams(needs_layout_passes=False),
    )
    def body(x_hbm, tgt_hbm, out_hbm, buf0, buf1, tgt_v, stage_v, sem0, sem1):
        wid = lax.axis_index("s") * _NUM_CORES + lax.axis_index("c")
        r0 = wid * rpt
        pltpu.sync_copy(tgt_hbm.at[pl.ds(r0, rpt)], tgt_v)
        ta = tgt_v[pl.ds(0, _L)]
        tb = tgt_v[pl.ds(_L, _L)]
        rows_a = jnp.arange(_L, dtype=jnp.int32)
        rows_b = rows_a + _L

        def chunk_src(c0):
            return x_hbm.at[pl.ds(r0, rpt), pl.ds(pl.multiple_of(c0, 128), _C)]

        def dense(buf, accs):
            def grp(k, accs):
                accs = list(accs)
                for r in range(rpt):
                    accs[r % _NACC] = accs[r % _NACC] + buf[r, pl.ds(k * _L, _L)]
                return tuple(accs)

            return lax.fori_loop(0, ngrp, grp, accs)

        def pick(buf, c0, t, rows, gacc):
            m = (t >= c0) & (t < c0 + _C)
            loc = jnp.where(m, t - c0, 0)
            vals = plsc.load_gather(buf, [rows, loc], mask=m)
            return gacc + jnp.where(m, vals, 0.0)

        def process(buf, c0, accs, gacc):
            accs = dense(buf, accs)
            gacc = pick(buf, c0, ta, rows_a, gacc)
            gacc = pick(buf, c0, tb, rows_b, gacc)
            return accs, gacc

        pltpu.async_copy(chunk_src(0), buf0, sem0)

        zero = jnp.zeros((_L,), jnp.float32)

        def pair(j, carry):
            accs, gacc = carry[:-1], carry[-1]
            c_even = j * (2 * _C)
            c_odd = c_even + _C
            pltpu.make_async_copy(chunk_src(c_even), buf0, sem0).wait()
            pltpu.async_copy(chunk_src(c_odd), buf1, sem1)
            accs, gacc = process(buf0, c_even, accs, gacc)
            pltpu.async_copy(chunk_src(c_even + 2 * _C), buf0, sem0)
            pltpu.make_async_copy(chunk_src(c_odd), buf1, sem1).wait()
            accs, gacc = process(buf1, c_odd, accs, gacc)
            return accs + (gacc,)

        carry = lax.fori_loop(0, npair, pair, (zero,) * _NACC + (zero,))
        accs, gacc = carry[:-1], carry[-1]
        c_last = (nchunk - 1) * _C
        pltpu.make_async_copy(chunk_src(c_last), buf0, sem0).wait()
        accs, gacc = process(buf0, c_last, accs, gacc)
        total = accs[0]
        for a in accs[1:]:
            total = total + a
        stage_v[...] = s_coef * total + g_coef * gacc
        pltpu.sync_copy(stage_v, out_hbm.at[wid])

    return body(x, tgt)


def _tc_tail_combine(part, x, tgt2d, aligned, k_total, s_coef, g_coef):
    n, v = x.shape
    ntail = v - aligned

    def body(part_ref, tail_ref, tgt_ref, out_ref):
        blk = tail_ref[...]
        lane = lax.broadcasted_iota(jnp.int32, (n, 128), 1)
        col = aligned + lane
        s_tail = jnp.sum(jnp.where(lane < ntail, blk, 0.0))
        g_tail = jnp.sum(jnp.where(col == tgt_ref[...], blk, 0.0))
        out_ref[0] = (
            k_total - jnp.sum(part_ref[...]) - s_coef * s_tail - g_coef * g_tail
        )

    return pl.pallas_call(
        body,
        grid=(1,),
        in_specs=[
            pl.BlockSpec(part.shape, lambda i: (0, 0)),
            pl.BlockSpec((n, 128), lambda i: (0, aligned // 128)),
            pl.BlockSpec((n, 1), lambda i: (0, 0)),
        ],
        out_specs=pl.BlockSpec(memory_space=pltpu.SMEM),
        out_shape=jax.ShapeDtypeStruct((1,), jnp.float32),
    )(part, x, tgt2d)


def kernel(x, target):
    n, v = x.shape
    s = _SMOOTHING / (v - 1)
    c = _CONFIDENCE
    k_total = n * ((v - 1) * s * math.log(s) + c * math.log(c))
    aligned = (v // 128) * 128 - ((v // 128) % 11) * 128

    tgt = target.astype(jnp.int32)
    part = _sc_partials(x, tgt, aligned, s, c - s)
    return _tc_tail_combine(part, x, tgt.reshape(n, 1), aligned, k_total, s, c - s)[0]
